# Initial kernel scaffold; baseline (speedup 1.0000x reference)
import jax
import jax.numpy as jnp
from jax.experimental import pallas as pl


def kernel(LL, LH, HL, HH, fm, conv_w, conv_b):
    raise NotImplementedError("write your pallas kernel here")


# speedup vs baseline: 1.3877x; 1.3877x over previous
import functools

import jax
import jax.numpy as jnp
import numpy as np
from jax.experimental import pallas as pl
from jax.experimental.pallas import tpu as pltpu


def _fused_kernel(ll_ref, lh_ref, hl_ref, hh_ref, fm_ref, w_ref, b_ref,
                  s0_ref, s1_ref, m_ref, o_ref, xf_ref, *,
                  C, Cf, Hh, Wh, R, GSZ):
    W = 2 * Wh
    H = 2 * Hh
    Cin = C + Cf
    B = W

    ll = ll_ref[0]
    lh = lh_ref[0]
    hl = hl_ref[0]
    hh = hh_ref[0]

    p = (ll - lh) * 0.5
    q = (ll + lh) * 0.5
    r = (hh - hl) * 0.5
    s = (hh + hl) * 0.5
    a = p + r
    b = p - r
    c = q - s
    d = q + s

    s0 = s0_ref[...]
    s1 = s1_ref[...]

    xf_ref[:, 0:B + W] = jnp.zeros((Cin, B + W), jnp.float32)
    xf_ref[:, B + (H + 1) * W:] = jnp.zeros((Cin, 2 * W), jnp.float32)

    for gi in range(Hh // GSZ):
        i0 = gi * GSZ
        sl = slice(i0 * Wh, (i0 + GSZ) * Wh)
        res_e = (jnp.dot(a[:, sl], s0, preferred_element_type=jnp.float32)
                 + jnp.dot(b[:, sl], s1, preferred_element_type=jnp.float32))
        res_o = (jnp.dot(c[:, sl], s0, preferred_element_type=jnp.float32)
                 + jnp.dot(d[:, sl], s1, preferred_element_type=jnp.float32))
        for m in range(GSZ):
            ye = 2 * (i0 + m)
            off_e = B + (ye + 1) * W
            xf_ref[0:C, off_e:off_e + W] = res_e[:, m * W:(m + 1) * W]
            xf_ref[0:C, off_e + W:off_e + 2 * W] = res_o[:, m * W:(m + 1) * W]

    xf_ref[C:Cin, B + W:B + (H + 1) * W] = fm_ref[0]

    bias = b_ref[...]
    mask_l = m_ref[0:1, :]
    mask_r = m_ref[1:2, :]

    for step in range(H // R):
        h0 = step * R
        acc = None
        for kh in range(3):
            row_off = B + (h0 + kh) * W
            for kw in range(3):
                off = row_off + (kw - 1)
                opnd = xf_ref[:, off:off + R * W]
                if kw == 0:
                    opnd = opnd * mask_l
                elif kw == 2:
                    opnd = opnd * mask_r
                t = jnp.dot(w_ref[3 * kh + kw], opnd,
                            preferred_element_type=jnp.float32)
                acc = t if acc is None else acc + t
        o_ref[0, :, h0 * W:(h0 + R) * W] = jnp.maximum(acc + bias, 0.0)


def kernel(LL, LH, HL, HH, fm, conv_w, conv_b):
    N, C, Hh, Wh = LL.shape
    H, W = 2 * Hh, 2 * Wh
    Nf, Cf, Hf, Wf = fm.shape
    assert (Nf, Hf, Wf) == (N, H, W)
    Cout, Cin, kh, kw = conv_w.shape
    assert (kh, kw) == (3, 3) and Cin == C + Cf

    ll2 = LL.reshape(N, C, Hh * Wh).astype(jnp.float32)
    lh2 = LH.reshape(N, C, Hh * Wh).astype(jnp.float32)
    hl2 = HL.reshape(N, C, Hh * Wh).astype(jnp.float32)
    hh2 = HH.reshape(N, C, Hh * Wh).astype(jnp.float32)
    fm2 = fm.reshape(N, Cf, H * W).astype(jnp.float32)

    w9 = (jnp.asarray(conv_w, jnp.float32)
          .transpose(2, 3, 0, 1).reshape(9, Cout, Cin))
    b2 = jnp.asarray(conv_b, jnp.float32).reshape(Cout, 1)

    GSZ = 4 if Hh % 4 == 0 else 1
    s0 = np.zeros((GSZ * Wh, GSZ * W), np.float32)
    s1 = np.zeros((GSZ * Wh, GSZ * W), np.float32)
    for m in range(GSZ):
        rows = m * Wh + np.arange(Wh)
        s0[rows, m * W + 2 * np.arange(Wh)] = 1.0
        s1[rows, m * W + 2 * np.arange(Wh) + 1] = 1.0
    s0 = jnp.asarray(s0)
    s1 = jnp.asarray(s1)

    R = 16 if H % 16 == 0 else (8 if H % 8 == 0 else H)

    lane = np.arange(R * W) % W
    masks = np.stack([(lane != 0).astype(np.float32),
                      (lane != W - 1).astype(np.float32)])
    masks = jnp.asarray(masks)

    kernel_fn = functools.partial(_fused_kernel, C=C, Cf=Cf, Hh=Hh, Wh=Wh,
                                  R=R, GSZ=GSZ)
    sub_spec = pl.BlockSpec((1, C, Hh * Wh), lambda n: (n, 0, 0))
    out = pl.pallas_call(
        kernel_fn,
        out_shape=jax.ShapeDtypeStruct((N, Cout, H * W), jnp.float32),
        grid=(N,),
        in_specs=[sub_spec, sub_spec, sub_spec, sub_spec,
                  pl.BlockSpec((1, Cf, H * W), lambda n: (n, 0, 0)),
                  pl.BlockSpec((9, Cout, Cin), lambda n: (0, 0, 0)),
                  pl.BlockSpec((Cout, 1), lambda n: (0, 0)),
                  pl.BlockSpec((GSZ * Wh, GSZ * W), lambda n: (0, 0)),
                  pl.BlockSpec((GSZ * Wh, GSZ * W), lambda n: (0, 0)),
                  pl.BlockSpec((2, R * W), lambda n: (0, 0))],
        out_specs=pl.BlockSpec((1, Cout, H * W), lambda n: (n, 0, 0)),
        scratch_shapes=[pltpu.VMEM((C + Cf, (H + 4) * W), jnp.float32)],
        compiler_params=pltpu.CompilerParams(
            dimension_semantics=("parallel",)),
    )(ll2, lh2, hl2, hh2, fm2, w9, b2, s0, s1, masks)
    return out.reshape(N, Cout, H, W)


# speedup vs baseline: 1.5395x; 1.1093x over previous
import functools

import jax
import jax.numpy as jnp
import numpy as np
from jax.experimental import pallas as pl
from jax.experimental.pallas import tpu as pltpu


def _fused_kernel(ll_ref, lh_ref, hl_ref, hh_ref, fm_ref, w_ref, b_ref,
                  t_ref, m_ref, o_ref, eo_ref, xf_ref, *,
                  C, Cf, Hh, Wh, R, CHK):
    W = 2 * Wh
    H = 2 * Hh
    Cin = C + Cf
    SCH = 2 * Hh + 2

    xf_ref[:, 0:W] = jnp.zeros((Cin, W), jnp.float32)
    xf_ref[:, (H + 1) * W:] = jnp.zeros((Cin, W), jnp.float32)

    sb_refs = (ll_ref, lh_ref, hl_ref, hh_ref)
    for ch in range(C // CHK):
        r0 = ch * CHK * Hh
        acc = None
        for s in range(4):
            x = sb_refs[s][0, r0:r0 + CHK * Hh, :]
            t = jnp.dot(x, t_ref[s * Wh:(s + 1) * Wh, :],
                        preferred_element_type=jnp.float32)
            acc = t if acc is None else acc + t
        for cl in range(CHK):
            rows = slice(cl * Hh, (cl + 1) * Hh)
            eo_ref[cl * SCH:cl * SCH + Hh, :] = acc[rows, 0:W]
            eo_ref[cl * SCH + Hh:cl * SCH + 2 * Hh, :] = acc[rows, W:2 * W]
        c0 = ch * CHK
        for i in range(Hh):
            off = (2 * i + 1) * W
            xf_ref[c0:c0 + CHK, off:off + W] = \
                eo_ref[i:i + CHK * SCH:SCH, :]
            xf_ref[c0:c0 + CHK, off + W:off + 2 * W] = \
                eo_ref[Hh + i:Hh + i + CHK * SCH:SCH, :]

    xf_ref[C:Cin, W:(H + 1) * W] = fm_ref[0]

    bias = b_ref[...]
    mask_l = m_ref[0:1, :]
    mask_r = m_ref[1:2, :]

    for step in range(H // R):
        h0 = step * R
        acc = None
        for kh in range(3):
            row_off = (h0 + kh) * W
            opnd = xf_ref[:, row_off:row_off + R * W]
            for kw in range(3):
                t = jnp.dot(w_ref[3 * kh + kw], opnd,
                            preferred_element_type=jnp.float32)
                if kw == 0:
                    t = jnp.roll(t, 1, axis=1) * mask_l
                elif kw == 2:
                    t = jnp.roll(t, -1, axis=1) * mask_r
                acc = t if acc is None else acc + t
        o_ref[0, :, h0 * W:(h0 + R) * W] = jnp.maximum(acc + bias, 0.0)


def kernel(LL, LH, HL, HH, fm, conv_w, conv_b):
    N, C, Hh, Wh = LL.shape
    H, W = 2 * Hh, 2 * Wh
    Nf, Cf, Hf, Wf = fm.shape
    assert (Nf, Hf, Wf) == (N, H, W)
    Cout, Cin, kh, kw = conv_w.shape
    assert (kh, kw) == (3, 3) and Cin == C + Cf

    ll2 = LL.reshape(N, C * Hh, Wh).astype(jnp.float32)
    lh2 = LH.reshape(N, C * Hh, Wh).astype(jnp.float32)
    hl2 = HL.reshape(N, C * Hh, Wh).astype(jnp.float32)
    hh2 = HH.reshape(N, C * Hh, Wh).astype(jnp.float32)
    fm2 = fm.reshape(N, Cf, H * W).astype(jnp.float32)

    w9 = (jnp.asarray(conv_w, jnp.float32)
          .transpose(2, 3, 0, 1).reshape(9, Cout, Cin))
    b2 = jnp.asarray(conv_b, jnp.float32).reshape(Cout, 1)

    coef = np.array([[.5, -.5, -.5, .5],
                     [.5, -.5, .5, -.5],
                     [.5, .5, -.5, -.5],
                     [.5, .5, .5, .5]], np.float32)
    T = np.zeros((4 * Wh, 2 * W), np.float32)
    j = np.arange(Wh)
    for s in range(4):
        T[s * Wh + j, 2 * j] = coef[0, s]
        T[s * Wh + j, 2 * j + 1] = coef[1, s]
        T[s * Wh + j, W + 2 * j] = coef[2, s]
        T[s * Wh + j, W + 2 * j + 1] = coef[3, s]
    T = jnp.asarray(T)

    R = 16 if H % 16 == 0 else (8 if H % 8 == 0 else H)
    CHK = max(1, min(C, 512 // Hh))
    while C % CHK:
        CHK -= 1

    lane = np.arange(R * W) % W
    masks = np.stack([(lane != 0).astype(np.float32),
                      (lane != W - 1).astype(np.float32)])
    masks = jnp.asarray(masks)

    kernel_fn = functools.partial(_fused_kernel, C=C, Cf=Cf, Hh=Hh, Wh=Wh,
                                  R=R, CHK=CHK)
    sub_spec = pl.BlockSpec((1, C * Hh, Wh), lambda n: (n, 0, 0))
    out = pl.pallas_call(
        kernel_fn,
        out_shape=jax.ShapeDtypeStruct((N, Cout, H * W), jnp.float32),
        grid=(N,),
        in_specs=[sub_spec, sub_spec, sub_spec, sub_spec,
                  pl.BlockSpec((1, Cf, H * W), lambda n: (n, 0, 0)),
                  pl.BlockSpec((9, Cout, Cin), lambda n: (0, 0, 0)),
                  pl.BlockSpec((Cout, 1), lambda n: (0, 0)),
                  pl.BlockSpec((4 * Wh, 2 * W), lambda n: (0, 0)),
                  pl.BlockSpec((2, R * W), lambda n: (0, 0))],
        out_specs=pl.BlockSpec((1, Cout, H * W), lambda n: (n, 0, 0)),
        scratch_shapes=[pltpu.VMEM((CHK * (2 * Hh + 2), W), jnp.float32),
                        pltpu.VMEM((C + Cf, (H + 2) * W), jnp.float32)],
        compiler_params=pltpu.CompilerParams(
            dimension_semantics=("parallel",)),
    )(ll2, lh2, hl2, hh2, fm2, w9, b2, T, masks)
    return out.reshape(N, Cout, H, W)
